# Initial kernel scaffold; baseline (speedup 1.0000x reference)
import functools

import jax
import jax.numpy as jnp
from jax import lax
from jax.experimental import pallas as pl
from jax.experimental.pallas import tpu as pltpu
from jax.experimental.pallas import tpu_sc as plsc

_NC = 2
_NS = 16
_NW = _NC * _NS
_CHUNK = 128
_K = 4
_ES = 32


def _gather_body(bi_hbm, ent_hbm, rel_hbm, head_out, rel_out, tail_out,
                 idx_v, h_buf, r_buf, t_buf, sem_h, sem_r, sem_t):
    nch = bi_hbm.shape[1] // _NW
    wid = lax.axis_index("s") * _NC + lax.axis_index("c")
    for j in range(3):
        for ch in range(nch):
            pltpu.sync_copy(bi_hbm.at[j, wid * nch + ch], idx_v.at[j, ch])
    for ch in range(nch):
        gb = (wid * nch + ch) * _CHUNK
        cp_h = pltpu.async_copy(ent_hbm.at[idx_v.at[0, ch]], h_buf, sem_h)
        cp_r = pltpu.async_copy(rel_hbm.at[idx_v.at[1, ch]], r_buf, sem_r)
        cp_t = pltpu.async_copy(ent_hbm.at[idx_v.at[2, ch]], t_buf, sem_t)
        cp_h.wait()
        pltpu.sync_copy(h_buf, head_out.at[pl.ds(gb, _CHUNK)])
        cp_r.wait()
        pltpu.sync_copy(r_buf, rel_out.at[pl.ds(gb, _CHUNK)])
        cp_t.wait()
        pltpu.sync_copy(t_buf, tail_out.at[pl.ds(gb, _CHUNK)])


def _sc_gather(bi3, entity_emb, relation_emb):
    b = bi3.shape[1] * _CHUNK
    de = entity_emb.shape[1]
    dr = relation_emb.shape[1]
    mesh = plsc.VectorSubcoreMesh(core_axis_name="c", subcore_axis_name="s",
                                  num_cores=_NC, num_subcores=_NS)
    return pl.kernel(
        _gather_body,
        out_type=(
            jax.ShapeDtypeStruct((b, de), jnp.float32),
            jax.ShapeDtypeStruct((b, dr), jnp.float32),
            jax.ShapeDtypeStruct((b, de), jnp.float32),
        ),
        mesh=mesh,
        scratch_types=[
            pltpu.VMEM((3, bi3.shape[1] // _NW, _CHUNK), jnp.int32),
            pltpu.VMEM((_CHUNK, de), jnp.float32),
            pltpu.VMEM((_CHUNK, dr), jnp.float32),
            pltpu.VMEM((_CHUNK, de), jnp.float32),
            pltpu.SemaphoreType.DMA,
            pltpu.SemaphoreType.DMA,
            pltpu.SemaphoreType.DMA,
        ],
    )(bi3, entity_emb, relation_emb)


def _attn_body(h_ref, r_ref, t_ref, wh_ref, wr_ref, wt_ref, b_ref,
               norm_ref, att_ref):
    h = h_ref[...]
    r = r_ref[...]
    t = t_ref[...]
    wh = wh_ref[...]
    wr = wr_ref[...]
    wt = wt_ref[...]
    bias = b_ref[0]
    rdot = jnp.sum(r * wr, axis=1, keepdims=True) + bias
    tmps = []
    for k in range(_K):
        hk = h[:, _ES * k:_ES * (k + 1)]
        tk = t[:, _ES * k:_ES * (k + 1)]
        dot = jnp.sum(hk * wh + tk * wt, axis=1, keepdims=True) + rdot
        tmps.append(jnp.maximum(dot, 0.0))
    m = jnp.maximum(jnp.maximum(tmps[0], tmps[1]), jnp.maximum(tmps[2], tmps[3]))
    es = [jnp.exp(tk - m) for tk in tmps]
    inv = 1.0 / (es[0] + es[1] + es[2] + es[3])
    x = r
    for k in range(_K):
        ak = es[k] * inv
        hk = h[:, _ES * k:_ES * (k + 1)]
        tk = t[:, _ES * k:_ES * (k + 1)]
        x = x + ak * (hk - tk)
    norm_ref[...] = jnp.sum(jnp.abs(x), axis=1)
    att_ref[...] = jnp.concatenate([e * inv for e in es], axis=1)


def _tc_attn(head, rel, tail, wh, wr, wt, bias):
    b = head.shape[0]
    blk = 2048
    grid = (b // blk,)
    return pl.pallas_call(
        _attn_body,
        grid=grid,
        in_specs=[
            pl.BlockSpec((blk, head.shape[1]), lambda i: (i, 0)),
            pl.BlockSpec((blk, rel.shape[1]), lambda i: (i, 0)),
            pl.BlockSpec((blk, tail.shape[1]), lambda i: (i, 0)),
            pl.BlockSpec((1, _ES), lambda i: (0, 0)),
            pl.BlockSpec((1, _ES), lambda i: (0, 0)),
            pl.BlockSpec((1, _ES), lambda i: (0, 0)),
            pl.BlockSpec(memory_space=pltpu.SMEM),
        ],
        out_specs=[
            pl.BlockSpec((blk,), lambda i: (i,)),
            pl.BlockSpec((blk, _K), lambda i: (i, 0)),
        ],
        out_shape=[
            jax.ShapeDtypeStruct((b,), jnp.float32),
            jax.ShapeDtypeStruct((b, _K), jnp.float32),
        ],
    )(head, rel, tail, wh, wr, wt, bias)


def kernel(batch_inputs, entity_emb, relation_emb, fc1_w, fc1_b):
    b = batch_inputs.shape[0]
    bi3 = batch_inputs.T.reshape(3, b // _CHUNK, _CHUNK)
    head, rel, tail = _sc_gather(bi3, entity_emb, relation_emb)
    wh = fc1_w[:, 0:_ES]
    wr = fc1_w[:, _ES:2 * _ES]
    wt = fc1_w[:, 2 * _ES:3 * _ES]
    norm, att = _tc_attn(head, rel, tail, wh, wr, wt, fc1_b)
    len_pos = b // 4
    pos_norm = jnp.tile(norm[:len_pos], (3,))
    neg_norm = norm[len_pos:]
    y = jnp.full((3 * len_pos,), -1.0, dtype=jnp.float32)
    return (pos_norm, neg_norm, y, att)


# speedup vs baseline: 1.0371x; 1.0371x over previous
import functools

import jax
import jax.numpy as jnp
from jax import lax
from jax.experimental import pallas as pl
from jax.experimental.pallas import tpu as pltpu
from jax.experimental.pallas import tpu_sc as plsc

_NC = 2
_NS = 16
_NW = _NC * _NS
_CHUNK = 128
_K = 4
_ES = 32


def _gather_body(bi_hbm, ent_hbm, rel4_hbm, head_out, rel_out, tail_out,
                 idx_v, h_buf, r_buf, t_buf, sem_h, sem_r, sem_t):
    nch = bi_hbm.shape[1] // _NW
    wid = lax.axis_index("s") * _NC + lax.axis_index("c")
    for j in range(3):
        for ch in range(nch):
            pltpu.sync_copy(bi_hbm.at[j, wid * nch + ch], idx_v.at[j, ch])
    for ch in range(nch):
        gb = (wid * nch + ch) * _CHUNK
        cp_h = pltpu.async_copy(ent_hbm.at[idx_v.at[0, ch]], h_buf, sem_h)
        cp_r = pltpu.async_copy(rel4_hbm.at[idx_v.at[1, ch]], r_buf, sem_r)
        cp_t = pltpu.async_copy(ent_hbm.at[idx_v.at[2, ch]], t_buf, sem_t)
        cp_h.wait()
        pltpu.sync_copy(h_buf, head_out.at[pl.ds(gb, _CHUNK)])
        cp_r.wait()
        pltpu.sync_copy(r_buf, rel_out.at[pl.ds(gb, _CHUNK)])
        cp_t.wait()
        pltpu.sync_copy(t_buf, tail_out.at[pl.ds(gb, _CHUNK)])


def _sc_gather(bi3, entity_emb, rel4):
    b = bi3.shape[1] * _CHUNK
    de = entity_emb.shape[1]
    dr = rel4.shape[1]
    mesh = plsc.VectorSubcoreMesh(core_axis_name="c", subcore_axis_name="s",
                                  num_cores=_NC, num_subcores=_NS)
    return pl.kernel(
        _gather_body,
        out_type=(
            jax.ShapeDtypeStruct((b, de), jnp.float32),
            jax.ShapeDtypeStruct((b, dr), jnp.float32),
            jax.ShapeDtypeStruct((b, de), jnp.float32),
        ),
        mesh=mesh,
        scratch_types=[
            pltpu.VMEM((3, bi3.shape[1] // _NW, _CHUNK), jnp.int32),
            pltpu.VMEM((_CHUNK, de), jnp.float32),
            pltpu.VMEM((_CHUNK, dr), jnp.float32),
            pltpu.VMEM((_CHUNK, de), jnp.float32),
            pltpu.SemaphoreType.DMA,
            pltpu.SemaphoreType.DMA,
            pltpu.SemaphoreType.DMA,
        ],
    )(bi3, entity_emb, rel4)


def _attn_body(h_ref, r4_ref, mod_ref, t_ref, wh_ref, wr_ref, wt_ref, b_ref,
               norm_ref, att_ref):
    h = h_ref[...]
    r4 = r4_ref[...]
    mod = mod_ref[...]
    t = t_ref[...]
    r = jnp.where(mod == 0, r4[:, 0:_ES], 0.0)
    r = r + jnp.where(mod == 1, r4[:, _ES:2 * _ES], 0.0)
    r = r + jnp.where(mod == 2, r4[:, 2 * _ES:3 * _ES], 0.0)
    r = r + jnp.where(mod == 3, r4[:, 3 * _ES:4 * _ES], 0.0)
    wh = wh_ref[...]
    wr = wr_ref[...]
    wt = wt_ref[...]
    bias = b_ref[0]
    rdot = jnp.sum(r * wr, axis=1, keepdims=True) + bias
    tmps = []
    for k in range(_K):
        hk = h[:, _ES * k:_ES * (k + 1)]
        tk = t[:, _ES * k:_ES * (k + 1)]
        dot = jnp.sum(hk * wh + tk * wt, axis=1, keepdims=True) + rdot
        tmps.append(jnp.maximum(dot, 0.0))
    m = jnp.maximum(jnp.maximum(tmps[0], tmps[1]), jnp.maximum(tmps[2], tmps[3]))
    es = [jnp.exp(tk - m) for tk in tmps]
    inv = 1.0 / (es[0] + es[1] + es[2] + es[3])
    x = r
    for k in range(_K):
        ak = es[k] * inv
        hk = h[:, _ES * k:_ES * (k + 1)]
        tk = t[:, _ES * k:_ES * (k + 1)]
        x = x + ak * (hk - tk)
    norm_ref[...] = jnp.sum(jnp.abs(x), axis=1)
    att_ref[...] = jnp.concatenate([e * inv for e in es], axis=1)


def _tc_attn(head, rel4g, rmod, tail, wh, wr, wt, bias):
    b = head.shape[0]
    blk = 2048
    grid = (b // blk,)
    return pl.pallas_call(
        _attn_body,
        grid=grid,
        in_specs=[
            pl.BlockSpec((blk, head.shape[1]), lambda i: (i, 0)),
            pl.BlockSpec((blk, rel4g.shape[1]), lambda i: (i, 0)),
            pl.BlockSpec((blk, 1), lambda i: (i, 0)),
            pl.BlockSpec((blk, tail.shape[1]), lambda i: (i, 0)),
            pl.BlockSpec((1, _ES), lambda i: (0, 0)),
            pl.BlockSpec((1, _ES), lambda i: (0, 0)),
            pl.BlockSpec((1, _ES), lambda i: (0, 0)),
            pl.BlockSpec(memory_space=pltpu.SMEM),
        ],
        out_specs=[
            pl.BlockSpec((blk,), lambda i: (i,)),
            pl.BlockSpec((blk, _K), lambda i: (i, 0)),
        ],
        out_shape=[
            jax.ShapeDtypeStruct((b,), jnp.float32),
            jax.ShapeDtypeStruct((b, _K), jnp.float32),
        ],
    )(head, rel4g, rmod, tail, wh, wr, wt, bias)


def kernel(batch_inputs, entity_emb, relation_emb, fc1_w, fc1_b):
    b = batch_inputs.shape[0]
    rows_per_r4 = 128 // relation_emb.shape[1]
    rel4 = relation_emb.reshape(relation_emb.shape[0] // rows_per_r4, 128)
    ridx = batch_inputs[:, 1]
    bi_t = jnp.stack([batch_inputs[:, 0], ridx // rows_per_r4, batch_inputs[:, 2]])
    bi3 = bi_t.reshape(3, b // _CHUNK, _CHUNK)
    head, rel4g, tail = _sc_gather(bi3, entity_emb, rel4)
    rmod = (ridx % rows_per_r4).astype(jnp.int32).reshape(b, 1)
    wh = fc1_w[:, 0:_ES]
    wr = fc1_w[:, _ES:2 * _ES]
    wt = fc1_w[:, 2 * _ES:3 * _ES]
    norm, att = _tc_attn(head, rel4g, rmod, tail, wh, wr, wt, fc1_b)
    len_pos = b // 4
    pos_norm = jnp.tile(norm[:len_pos], (3,))
    neg_norm = norm[len_pos:]
    y = jnp.full((3 * len_pos,), -1.0, dtype=jnp.float32)
    return (pos_norm, neg_norm, y, att)
